# baseline (speedup 1.0000x reference)
import functools

import jax
import jax.numpy as jnp
from jax import lax
from jax.experimental import pallas as pl
from jax.experimental.pallas import tpu as pltpu
from jax.experimental.pallas import tpu_sc as plsc

_NC = 2
_NS = 16
_NW = _NC * _NS
_L = 16
_K = 128
_IB = 8
_BN = 1000
_ALIGN = 16384
_DUMP = 2048


def _enc_body(pw_ref, x_ref, wt_ref, h_ref):
    xb = x_ref[...]
    pw = pw_ref[0, 0]
    xa = jnp.where(xb > 0, xb, pw * xb)
    h_ref[...] = jnp.dot(xa, wt_ref[...], preferred_element_type=jnp.float32)


def _mlp_body(p_ref, h_ref, c_ref, embp_ref, w1t_ref, b1_ref, w2t_ref,
              b2_ref, out_ref):
    a = p_ref[0] + h_ref[...]
    embp = embp_ref[...]
    a = a + jnp.dot(c_ref[...], embp, preferred_element_type=jnp.float32)
    a = a + embp[12:13, :]
    hid = jnp.dot(a, w1t_ref[...], preferred_element_type=jnp.float32)
    hid = jnp.maximum(hid + b1_ref[...], 0.0)
    out_ref[...] = (jnp.dot(hid, w2t_ref[...],
                            preferred_element_type=jnp.float32) + b2_ref[...])


def _make_part_kernel(n_chunks_w, cap_alloc):
    mesh = plsc.VectorSubcoreMesh(core_axis_name="c", subcore_axis_name="s")
    n_blocks = n_chunks_w // _IB
    cap_pt = cap_alloc // _NS

    @functools.partial(
        pl.kernel,
        mesh=mesh,
        out_type=jax.ShapeDtypeStruct((_NC, cap_alloc), jnp.int32),
        scratch_types=[
            pltpu.VMEM((_IB, _K), jnp.int32),
            pltpu.VMEM((_IB, _K), jnp.int32),
            pltpu.VMEM_SHARED((cap_alloc,), jnp.int32),
        ],
    )
    def part_kernel(posp_hbm, recp_hbm, ztrash_hbm, out_hbm, pos_v, rec_v,
                    part_sh):
        cid = lax.axis_index("c")
        sid = lax.axis_index("s")
        wid = sid * _NC + cid
        pltpu.sync_copy(ztrash_hbm.at[pl.ds(sid * cap_pt, cap_pt)],
                        part_sh.at[pl.ds(sid * cap_pt, cap_pt)])
        plsc.subcore_barrier()

        def blk(b, carry):
            pltpu.sync_copy(posp_hbm.at[wid].at[pl.ds(b * _IB, _IB)], pos_v)
            pltpu.sync_copy(recp_hbm.at[wid].at[pl.ds(b * _IB, _IB)], rec_v)
            for j in range(_IB):
                pltpu.sync_copy(rec_v.at[j], part_sh.at[pos_v.at[j]])
            return carry

        lax.fori_loop(0, n_blocks, blk, 0)
        plsc.subcore_barrier()
        pltpu.sync_copy(part_sh.at[pl.ds(sid * cap_pt, cap_pt)],
                        out_hbm.at[cid].at[pl.ds(sid * cap_pt, cap_pt)])

    return part_kernel


def _make_aggr_kernel(cap_chunks, nh, cl, d):
    rows_pt = nh // _NS
    cnt_pt = cl // _NS
    mesh = plsc.VectorSubcoreMesh(core_axis_name="c", subcore_axis_name="s")

    @functools.partial(
        pl.kernel,
        mesh=mesh,
        out_type=[
            jax.ShapeDtypeStruct((_NC, nh, d), jnp.float32),
            jax.ShapeDtypeStruct((_NC, cl), jnp.float32),
        ],
        scratch_types=[
            pltpu.VMEM((_IB, _K), jnp.int32),
            pltpu.VMEM((_IB, _K), jnp.int32),
            pltpu.VMEM((_IB, _K), jnp.int32),
            pltpu.VMEM((2, _K, d), jnp.float32),
            pltpu.VMEM((_K,), jnp.float32),
            pltpu.VMEM((16,), jnp.int32),
            pltpu.VMEM((_K,), jnp.int32),
            pltpu.VMEM_SHARED((nh, d), jnp.float32),
            pltpu.VMEM_SHARED((nh, d), jnp.float32),
            pltpu.VMEM_SHARED((cl,), jnp.float32),
            pltpu.SemaphoreType.DMA,
            pltpu.SemaphoreType.DMA,
            pltpu.SemaphoreType.DMA,
            pltpu.SemaphoreType.DMA,
        ],
    )
    def aggr_kernel(hp_hbm, srcp_hbm, dstp_hbm, cntp_hbm, offc_hbm,
                    zrow_hbm, zcnt_hbm, aggr_out, cnt_out,
                    src_v, dst_v, cidx_v, rows_v, ones_v, off_v, tidx_v,
                    h_sh, aggr_sh, cnt_sh, sem0, sem1, ssem0, ssem1):
        sems = (sem0, sem1)
        ssems = (ssem0, ssem1)
        cid = lax.axis_index("c")
        sid = lax.axis_index("s")
        pltpu.sync_copy(zrow_hbm.at[pl.ds(sid * rows_pt, rows_pt)],
                        aggr_sh.at[pl.ds(sid * rows_pt, rows_pt)])
        pltpu.sync_copy(zcnt_hbm.at[pl.ds(sid * cnt_pt, cnt_pt)],
                        cnt_sh.at[pl.ds(sid * cnt_pt, cnt_pt)])
        pltpu.sync_copy(offc_hbm, off_v)
        offv = off_v[...]
        offsc = [offv[k] for k in range(5)]
        for t in range(_K // _L):
            ones_v[pl.ds(t * _L, _L)] = jnp.ones((_L,), jnp.float32)
            tidx_v[pl.ds(t * _L, _L)] = lax.iota(jnp.int32, _L) + (nh - _L)

        for p in range(2):
            plsc.subcore_barrier()
            pltpu.sync_copy(hp_hbm.at[p].at[pl.ds(sid * rows_pt, rows_pt)],
                            h_sh.at[pl.ds(sid * rows_pt, rows_pt)])
            plsc.subcore_barrier()
            lo = jnp.where(cid == 0, offsc[p], offsc[2 + p])
            hi = jnp.where(cid == 0, offsc[p + 1], offsc[p + 3])
            per = (hi - lo) // 16
            my_lo = lo + sid * per
            for q in range(2):
                pltpu.async_copy(rows_v.at[q], aggr_sh.at[tidx_v], ssems[q],
                                 add=True)

            def blk(b, carry):
                base = pl.multiple_of(my_lo + b * _IB, _IB)
                pltpu.sync_copy(srcp_hbm.at[pl.ds(base, _IB)], src_v)
                pltpu.sync_copy(dstp_hbm.at[pl.ds(base, _IB)], dst_v)
                pltpu.sync_copy(cntp_hbm.at[pl.ds(base, _IB)], cidx_v)
                pltpu.make_async_copy(rows_v.at[0], aggr_sh.at[tidx_v],
                                      ssems[0]).wait()
                pltpu.async_copy(h_sh.at[src_v.at[0]], rows_v.at[0], sems[0])
                for j in range(_IB):
                    if j + 1 < _IB:
                        pltpu.make_async_copy(rows_v.at[(j + 1) % 2],
                                              aggr_sh.at[tidx_v],
                                              ssems[(j + 1) % 2]).wait()
                        pltpu.async_copy(h_sh.at[src_v.at[j + 1]],
                                         rows_v.at[(j + 1) % 2],
                                         sems[(j + 1) % 2])
                    pltpu.make_async_copy(h_sh.at[src_v.at[j]],
                                          rows_v.at[j % 2],
                                          sems[j % 2]).wait()
                    pltpu.async_copy(rows_v.at[j % 2],
                                     aggr_sh.at[dst_v.at[j]], ssems[j % 2],
                                     add=True)
                    pltpu.sync_copy(ones_v, cnt_sh.at[cidx_v.at[j]], add=True)
                return carry

            lax.fori_loop(0, per // _IB, blk, 0)
            for q in range(2):
                pltpu.make_async_copy(rows_v.at[q], aggr_sh.at[tidx_v],
                                      ssems[q]).wait()

        plsc.subcore_barrier()
        pltpu.sync_copy(aggr_sh.at[pl.ds(sid * rows_pt, rows_pt)],
                        aggr_out.at[cid].at[pl.ds(sid * rows_pt, rows_pt)])
        pltpu.sync_copy(cnt_sh.at[pl.ds(sid * cnt_pt, cnt_pt)],
                        cnt_out.at[cid].at[pl.ds(sid * cnt_pt, cnt_pt)])

    return aggr_kernel


def kernel(x, edge_index, edge_attr, prelu_w, W_enc, emb1, emb2, W1, b1, W2, b2):
    n, d = x.shape
    e = edge_index.shape[1]
    f32, i32, u32 = jnp.float32, jnp.int32, jnp.uint32
    n2 = n // 2
    nh = -(-(n2 + 1) // (_NS * 8)) * (_NS * 8)
    cl = -(-((n2 + 1) * 18) // (_NS * 128)) * (_NS * 128)

    grid = n // _BN
    h = pl.pallas_call(
        _enc_body,
        grid=(grid,),
        in_specs=[
            pl.BlockSpec((1, 1), lambda i: (0, 0)),
            pl.BlockSpec((_BN, d), lambda i: (i, 0)),
            pl.BlockSpec((d, d), lambda i: (0, 0)),
        ],
        out_specs=pl.BlockSpec((_BN, d), lambda i: (i, 0)),
        out_shape=jax.ShapeDtypeStruct((n, d), f32),
    )(prelu_w.reshape(1, 1), x, W_enc.T)

    src = edge_index[0]
    dst = edge_index[1]
    code = edge_attr[:, 0] * 3 + edge_attr[:, 1]
    src_h = (src >= n2).astype(i32)
    dst_h = (dst >= n2).astype(i32)
    dst_l = dst - dst_h * n2
    rec = lax.bitcast_convert_type(
        (src.astype(u32) << 18) | (code.astype(u32) << 13)
        | dst_l.astype(u32), i32)
    grp = dst_h * 2 + src_h
    onehot = (grp[None, :] == jnp.arange(4, dtype=i32)[:, None])
    ranks = jnp.cumsum(onehot.astype(i32), axis=1) - 1
    counts = ranks[:, -1] + 1
    ends = jnp.cumsum(-(-counts // _ALIGN) * _ALIGN)
    offs = jnp.concatenate([jnp.zeros((1,), i32), ends.astype(i32)])
    pos = jnp.zeros((e,), i32)
    for k in range(4):
        pos = jnp.where(grp == k, ranks[k] + offs[k], pos)

    cap_total = e + 4 * _ALIGN
    cap_alloc = -(-(cap_total + _DUMP) // (_NS * 128)) * (_NS * 128)
    rec_trash = n2
    fill_cand = (offs[:4] + counts)[:, None] + jnp.arange(_ALIGN, dtype=i32)
    fill_valid = fill_cand < offs[1:][:, None]
    dump_ids = cap_total + (
        jnp.arange(4 * _ALIGN, dtype=i32) % _DUMP).reshape(4, _ALIGN)
    fill_pos = jnp.where(fill_valid, fill_cand, dump_ids).reshape(-1)

    n_entries = e + 4 * _ALIGN
    n_chunks_w = -(-n_entries // (_NW * _K * _IB)) * _IB
    e_pad_a = _NW * _K * n_chunks_w
    pad_a = e_pad_a - n_entries
    pos_all = jnp.concatenate(
        [pos, fill_pos,
         cap_total + jnp.arange(pad_a, dtype=i32) % _DUMP])
    rec_all = jnp.concatenate(
        [rec, jnp.full((4 * _ALIGN + pad_a,), rec_trash, i32)])
    posp = pos_all.reshape(_NW, n_chunks_w, _K)
    recp = rec_all.reshape(_NW, n_chunks_w, _K)

    ztrash = jnp.full((cap_alloc,), rec_trash, i32)
    recs2 = _make_part_kernel(n_chunks_w, cap_alloc)(posp, recp, ztrash)
    recs = jnp.where(recs2[0] != rec_trash, recs2[0], recs2[1])

    ur = lax.bitcast_convert_type(recs, u32)
    src_g = (ur >> 18).astype(i32)
    code_u = ((ur >> 13) & 31).astype(i32)
    dst_u = jnp.minimum((ur & 8191).astype(i32), n2)
    src_u = jnp.minimum(src_g - (src_g >= n2).astype(i32) * n2, nh - 1)
    cidx_u = dst_u * 18 + code_u
    cap_chunks = cap_alloc // _K
    srcp2 = src_u.reshape(cap_chunks, _K)
    dstp2 = dst_u.reshape(cap_chunks, _K)
    cntp2 = cidx_u.reshape(cap_chunks, _K)
    offc = jnp.concatenate([offs // _K, jnp.zeros((11,), i32)])

    hp = jnp.zeros((2, nh, d), f32)
    hp = hp.at[0, :n2].set(lax.slice(h, (0, 0), (n2, d)))
    hp = hp.at[1, :n2].set(lax.slice(h, (n2, 0), (n, d)))
    zrow = jnp.zeros((nh, d), f32)
    zcnt = jnp.zeros((cl,), f32)

    aggr2, cnt2 = _make_aggr_kernel(cap_chunks, nh, cl, d)(
        hp, srcp2, dstp2, cntp2, offc, zrow, zcnt)

    c0 = lax.slice(cnt2[0], (0,), (n2 * 18,)).reshape(n2, 18)
    c1 = lax.slice(cnt2[1], (0,), (n2 * 18,)).reshape(n2, 18)
    call = jnp.concatenate([c0, c1])

    emb18 = (emb1[:, None, :] + emb2[None, :, :]).reshape(18, d)
    nb_half = n2 // _BN

    out = pl.pallas_call(
        _mlp_body,
        grid=(grid,),
        in_specs=[
            pl.BlockSpec((1, _BN, d),
                         lambda i: (i // nb_half, i % nb_half, 0)),
            pl.BlockSpec((_BN, d), lambda i: (i, 0)),
            pl.BlockSpec((_BN, 18), lambda i: (i, 0)),
            pl.BlockSpec((18, d), lambda i: (0, 0)),
            pl.BlockSpec((d, 2 * d), lambda i: (0, 0)),
            pl.BlockSpec((1, 2 * d), lambda i: (0, 0)),
            pl.BlockSpec((2 * d, d), lambda i: (0, 0)),
            pl.BlockSpec((1, d), lambda i: (0, 0)),
        ],
        out_specs=pl.BlockSpec((_BN, d), lambda i: (i, 0)),
        out_shape=jax.ShapeDtypeStruct((n, d), f32),
    )(aggr2, h, call, emb18, W1.T, b1.reshape(1, -1), W2.T,
      b2.reshape(1, -1))
    return out


# speedup vs baseline: 1.0204x; 1.0204x over previous
import functools

import jax
import jax.numpy as jnp
from jax import lax
from jax.experimental import pallas as pl
from jax.experimental.pallas import tpu as pltpu
from jax.experimental.pallas import tpu_sc as plsc

_NC = 2
_NS = 16
_NW = _NC * _NS
_L = 16
_K = 128
_IB = 8
_BN = 1000
_ALIGN = 16384
_DUMP = 2048


def _enc_body(pw_ref, x_ref, wt_ref, h_ref):
    xb = x_ref[...]
    pw = pw_ref[0, 0]
    xa = jnp.where(xb > 0, xb, pw * xb)
    h_ref[...] = jnp.dot(xa, wt_ref[...], preferred_element_type=jnp.float32)


def _mlp_body(p_ref, h_ref, c_ref, embp_ref, w1t_ref, b1_ref, w2t_ref,
              b2_ref, out_ref):
    a = p_ref[0] + h_ref[...]
    embp = embp_ref[...]
    a = a + jnp.dot(c_ref[...], embp, preferred_element_type=jnp.float32)
    a = a + embp[12:13, :]
    hid = jnp.dot(a, w1t_ref[...], preferred_element_type=jnp.float32)
    hid = jnp.maximum(hid + b1_ref[...], 0.0)
    out_ref[...] = (jnp.dot(hid, w2t_ref[...],
                            preferred_element_type=jnp.float32) + b2_ref[...])


def _make_part_kernel(n_chunks_w, cap_alloc):
    mesh = plsc.VectorSubcoreMesh(core_axis_name="c", subcore_axis_name="s")
    n_blocks = n_chunks_w // _IB
    cap_pt = cap_alloc // _NS

    @functools.partial(
        pl.kernel,
        mesh=mesh,
        out_type=jax.ShapeDtypeStruct((_NC, cap_alloc), jnp.int32),
        scratch_types=[
            pltpu.VMEM((_IB, _K), jnp.int32),
            pltpu.VMEM((_IB, _K), jnp.int32),
            pltpu.VMEM_SHARED((cap_alloc,), jnp.int32),
        ],
    )
    def part_kernel(posp_hbm, recp_hbm, ztrash_hbm, out_hbm, pos_v, rec_v,
                    part_sh):
        cid = lax.axis_index("c")
        sid = lax.axis_index("s")
        wid = sid * _NC + cid
        pltpu.sync_copy(ztrash_hbm.at[pl.ds(sid * cap_pt, cap_pt)],
                        part_sh.at[pl.ds(sid * cap_pt, cap_pt)])
        plsc.subcore_barrier()

        def blk(b, carry):
            pltpu.sync_copy(posp_hbm.at[wid].at[pl.ds(b * _IB, _IB)], pos_v)
            pltpu.sync_copy(recp_hbm.at[wid].at[pl.ds(b * _IB, _IB)], rec_v)
            for j in range(_IB):
                pltpu.sync_copy(rec_v.at[j], part_sh.at[pos_v.at[j]])
            return carry

        lax.fori_loop(0, n_blocks, blk, 0)
        plsc.subcore_barrier()
        pltpu.sync_copy(part_sh.at[pl.ds(sid * cap_pt, cap_pt)],
                        out_hbm.at[cid].at[pl.ds(sid * cap_pt, cap_pt)])

    return part_kernel


def _make_aggr_kernel(cap_chunks, nh, cl, d):
    rows_pt = nh // _NS
    cnt_pt = cl // _NS
    mesh = plsc.VectorSubcoreMesh(core_axis_name="c", subcore_axis_name="s")

    @functools.partial(
        pl.kernel,
        mesh=mesh,
        out_type=[
            jax.ShapeDtypeStruct((_NC, nh, d), jnp.float32),
            jax.ShapeDtypeStruct((_NC, cl), jnp.float32),
        ],
        scratch_types=[
            pltpu.VMEM((_IB, _K), jnp.int32),
            pltpu.VMEM((_IB, _K), jnp.int32),
            pltpu.VMEM((_IB, _K), jnp.int32),
            pltpu.VMEM((2, _K, d), jnp.float32),
            pltpu.VMEM((_K,), jnp.float32),
            pltpu.VMEM((16,), jnp.int32),
            pltpu.VMEM_SHARED((nh, d), jnp.float32),
            pltpu.VMEM_SHARED((nh, d), jnp.float32),
            pltpu.VMEM_SHARED((cl,), jnp.float32),
            pltpu.SemaphoreType.DMA,
            pltpu.SemaphoreType.DMA,
        ],
    )
    def aggr_kernel(hp_hbm, srcp_hbm, dstp_hbm, cntp_hbm, offc_hbm,
                    zrow_hbm, zcnt_hbm, aggr_out, cnt_out,
                    src_v, dst_v, cidx_v, rows_v, ones_v, off_v,
                    h_sh, aggr_sh, cnt_sh, sem0, sem1):
        sems = (sem0, sem1)
        cid = lax.axis_index("c")
        sid = lax.axis_index("s")
        pltpu.sync_copy(zrow_hbm.at[pl.ds(sid * rows_pt, rows_pt)],
                        aggr_sh.at[pl.ds(sid * rows_pt, rows_pt)])
        pltpu.sync_copy(zcnt_hbm.at[pl.ds(sid * cnt_pt, cnt_pt)],
                        cnt_sh.at[pl.ds(sid * cnt_pt, cnt_pt)])
        pltpu.sync_copy(offc_hbm, off_v)
        offv = off_v[...]
        offsc = [offv[k] for k in range(5)]
        for t in range(_K // _L):
            ones_v[pl.ds(t * _L, _L)] = jnp.ones((_L,), jnp.float32)

        for p in range(2):
            plsc.subcore_barrier()
            pltpu.sync_copy(hp_hbm.at[p].at[pl.ds(sid * rows_pt, rows_pt)],
                            h_sh.at[pl.ds(sid * rows_pt, rows_pt)])
            plsc.subcore_barrier()
            lo = jnp.where(cid == 0, offsc[p], offsc[2 + p])
            hi = jnp.where(cid == 0, offsc[p + 1], offsc[p + 3])
            per = (hi - lo) // 16
            my_lo = lo + sid * per

            def blk(b, carry):
                base = pl.multiple_of(my_lo + b * _IB, _IB)
                pltpu.sync_copy(srcp_hbm.at[pl.ds(base, _IB)], src_v)
                pltpu.sync_copy(dstp_hbm.at[pl.ds(base, _IB)], dst_v)
                pltpu.sync_copy(cntp_hbm.at[pl.ds(base, _IB)], cidx_v)
                pltpu.async_copy(h_sh.at[src_v.at[0]], rows_v.at[0], sems[0])
                for j in range(_IB):
                    if j + 1 < _IB:
                        pltpu.async_copy(h_sh.at[src_v.at[j + 1]],
                                         rows_v.at[(j + 1) % 2],
                                         sems[(j + 1) % 2])
                    pltpu.make_async_copy(h_sh.at[src_v.at[j]],
                                          rows_v.at[j % 2],
                                          sems[j % 2]).wait()
                    pltpu.sync_copy(rows_v.at[j % 2],
                                    aggr_sh.at[dst_v.at[j]], add=True)
                    pltpu.sync_copy(ones_v, cnt_sh.at[cidx_v.at[j]], add=True)
                return carry

            lax.fori_loop(0, per // _IB, blk, 0)

        plsc.subcore_barrier()
        pltpu.sync_copy(aggr_sh.at[pl.ds(sid * rows_pt, rows_pt)],
                        aggr_out.at[cid].at[pl.ds(sid * rows_pt, rows_pt)])
        pltpu.sync_copy(cnt_sh.at[pl.ds(sid * cnt_pt, cnt_pt)],
                        cnt_out.at[cid].at[pl.ds(sid * cnt_pt, cnt_pt)])

    return aggr_kernel


def kernel(x, edge_index, edge_attr, prelu_w, W_enc, emb1, emb2, W1, b1, W2, b2):
    n, d = x.shape
    e = edge_index.shape[1]
    f32, i32, u32 = jnp.float32, jnp.int32, jnp.uint32
    n2 = n // 2
    nh = -(-(n2 + 1) // (_NS * 8)) * (_NS * 8)
    cl = -(-((n2 + 1) * 18) // (_NS * 128)) * (_NS * 128)

    grid = n // _BN
    h = pl.pallas_call(
        _enc_body,
        grid=(grid,),
        in_specs=[
            pl.BlockSpec((1, 1), lambda i: (0, 0)),
            pl.BlockSpec((_BN, d), lambda i: (i, 0)),
            pl.BlockSpec((d, d), lambda i: (0, 0)),
        ],
        out_specs=pl.BlockSpec((_BN, d), lambda i: (i, 0)),
        out_shape=jax.ShapeDtypeStruct((n, d), f32),
    )(prelu_w.reshape(1, 1), x, W_enc.T)

    src = edge_index[0]
    dst = edge_index[1]
    code = edge_attr[:, 0] * 3 + edge_attr[:, 1]
    src_h = (src >= n2).astype(i32)
    dst_h = (dst >= n2).astype(i32)
    dst_l = dst - dst_h * n2
    rec = lax.bitcast_convert_type(
        (src.astype(u32) << 18) | (code.astype(u32) << 13)
        | dst_l.astype(u32), i32)
    grp = dst_h * 2 + src_h
    onehot = (grp[None, :] == jnp.arange(4, dtype=i32)[:, None])
    ranks = jnp.cumsum(onehot.astype(i32), axis=1) - 1
    counts = ranks[:, -1] + 1
    ends = jnp.cumsum(-(-counts // _ALIGN) * _ALIGN)
    offs = jnp.concatenate([jnp.zeros((1,), i32), ends.astype(i32)])
    pos = jnp.zeros((e,), i32)
    for k in range(4):
        pos = jnp.where(grp == k, ranks[k] + offs[k], pos)

    cap_total = e + 4 * _ALIGN
    cap_alloc = -(-(cap_total + _DUMP) // (_NS * 128)) * (_NS * 128)
    rec_trash = n2
    fill_cand = (offs[:4] + counts)[:, None] + jnp.arange(_ALIGN, dtype=i32)
    fill_valid = fill_cand < offs[1:][:, None]
    dump_ids = cap_total + (
        jnp.arange(4 * _ALIGN, dtype=i32) % _DUMP).reshape(4, _ALIGN)
    fill_pos = jnp.where(fill_valid, fill_cand, dump_ids).reshape(-1)

    n_entries = e + 4 * _ALIGN
    n_chunks_w = -(-n_entries // (_NW * _K * _IB)) * _IB
    e_pad_a = _NW * _K * n_chunks_w
    pad_a = e_pad_a - n_entries
    pos_all = jnp.concatenate(
        [pos, fill_pos,
         cap_total + jnp.arange(pad_a, dtype=i32) % _DUMP])
    rec_all = jnp.concatenate(
        [rec, jnp.full((4 * _ALIGN + pad_a,), rec_trash, i32)])
    posp = pos_all.reshape(_NW, n_chunks_w, _K)
    recp = rec_all.reshape(_NW, n_chunks_w, _K)

    ztrash = jnp.full((cap_alloc,), rec_trash, i32)
    recs2 = _make_part_kernel(n_chunks_w, cap_alloc)(posp, recp, ztrash)
    recs = jnp.where(recs2[0] != rec_trash, recs2[0], recs2[1])

    ur = lax.bitcast_convert_type(recs, u32)
    src_g = (ur >> 18).astype(i32)
    code_u = ((ur >> 13) & 31).astype(i32)
    dst_u = jnp.minimum((ur & 8191).astype(i32), n2)
    src_u = jnp.minimum(src_g - (src_g >= n2).astype(i32) * n2, nh - 1)
    cidx_u = dst_u * 18 + code_u
    cap_chunks = cap_alloc // _K
    srcp2 = src_u.reshape(cap_chunks, _K)
    dstp2 = dst_u.reshape(cap_chunks, _K)
    cntp2 = cidx_u.reshape(cap_chunks, _K)
    offc = jnp.concatenate([offs // _K, jnp.zeros((11,), i32)])

    hp = jnp.zeros((2, nh, d), f32)
    hp = hp.at[0, :n2].set(lax.slice(h, (0, 0), (n2, d)))
    hp = hp.at[1, :n2].set(lax.slice(h, (n2, 0), (n, d)))
    zrow = jnp.zeros((nh, d), f32)
    zcnt = jnp.zeros((cl,), f32)

    aggr2, cnt2 = _make_aggr_kernel(cap_chunks, nh, cl, d)(
        hp, srcp2, dstp2, cntp2, offc, zrow, zcnt)

    c0 = lax.slice(cnt2[0], (0,), (n2 * 18,)).reshape(n2, 18)
    c1 = lax.slice(cnt2[1], (0,), (n2 * 18,)).reshape(n2, 18)
    call = jnp.concatenate([c0, c1])

    emb18 = (emb1[:, None, :] + emb2[None, :, :]).reshape(18, d)
    nb_half = n2 // _BN

    out = pl.pallas_call(
        _mlp_body,
        grid=(grid,),
        in_specs=[
            pl.BlockSpec((1, _BN, d),
                         lambda i: (i // nb_half, i % nb_half, 0)),
            pl.BlockSpec((_BN, d), lambda i: (i, 0)),
            pl.BlockSpec((_BN, 18), lambda i: (i, 0)),
            pl.BlockSpec((18, d), lambda i: (0, 0)),
            pl.BlockSpec((d, 2 * d), lambda i: (0, 0)),
            pl.BlockSpec((1, 2 * d), lambda i: (0, 0)),
            pl.BlockSpec((2 * d, d), lambda i: (0, 0)),
            pl.BlockSpec((1, d), lambda i: (0, 0)),
        ],
        out_specs=pl.BlockSpec((_BN, d), lambda i: (i, 0)),
        out_shape=jax.ShapeDtypeStruct((n, d), f32),
    )(aggr2, h, call, emb18, W1.T, b1.reshape(1, -1), W2.T,
      b2.reshape(1, -1))
    return out


# speedup vs baseline: 1.0589x; 1.0377x over previous
import functools

import jax
import jax.numpy as jnp
from jax import lax
from jax.experimental import pallas as pl
from jax.experimental.pallas import tpu as pltpu
from jax.experimental.pallas import tpu_sc as plsc

_NC = 2
_NS = 16
_NW = _NC * _NS
_L = 16
_K = 128
_IB = 8
_BN = 1000
_ALIGN = 16384
_DUMP = 2048


def _enc_body(pw_ref, x_ref, wt_ref, h_ref):
    xb = x_ref[...]
    pw = pw_ref[0, 0]
    xa = jnp.where(xb > 0, xb, pw * xb)
    h_ref[...] = jnp.dot(xa, wt_ref[...], preferred_element_type=jnp.float32)


def _mlp_body(p_ref, h_ref, c_ref, embp_ref, w1t_ref, b1_ref, w2t_ref,
              b2_ref, out_ref):
    a = p_ref[0] + h_ref[...]
    embp = embp_ref[...]
    a = a + jnp.dot(c_ref[...], embp, preferred_element_type=jnp.float32)
    a = a + embp[12:13, :]
    hid = jnp.dot(a, w1t_ref[...], preferred_element_type=jnp.float32)
    hid = jnp.maximum(hid + b1_ref[...], 0.0)
    out_ref[...] = (jnp.dot(hid, w2t_ref[...],
                            preferred_element_type=jnp.float32) + b2_ref[...])


def _make_part_kernel(n_chunks_w, cap_alloc):
    mesh = plsc.VectorSubcoreMesh(core_axis_name="c", subcore_axis_name="s")
    n_blocks = n_chunks_w // _IB
    cap_pt = cap_alloc // _NS

    @functools.partial(
        pl.kernel,
        mesh=mesh,
        out_type=jax.ShapeDtypeStruct((_NC, cap_alloc), jnp.int32),
        scratch_types=[
            pltpu.VMEM((_IB, _K), jnp.int32),
            pltpu.VMEM((_IB, _K), jnp.int32),
            pltpu.VMEM_SHARED((cap_alloc,), jnp.int32),
        ],
    )
    def part_kernel(posp_hbm, recp_hbm, ztrash_hbm, out_hbm, pos_v, rec_v,
                    part_sh):
        cid = lax.axis_index("c")
        sid = lax.axis_index("s")
        wid = sid * _NC + cid
        pltpu.sync_copy(ztrash_hbm.at[pl.ds(sid * cap_pt, cap_pt)],
                        part_sh.at[pl.ds(sid * cap_pt, cap_pt)])
        plsc.subcore_barrier()

        def blk(b, carry):
            pltpu.sync_copy(posp_hbm.at[wid].at[pl.ds(b * _IB, _IB)], pos_v)
            pltpu.sync_copy(recp_hbm.at[wid].at[pl.ds(b * _IB, _IB)], rec_v)
            for j in range(_IB):
                pltpu.sync_copy(rec_v.at[j], part_sh.at[pos_v.at[j]])
            return carry

        lax.fori_loop(0, n_blocks, blk, 0)
        plsc.subcore_barrier()
        pltpu.sync_copy(part_sh.at[pl.ds(sid * cap_pt, cap_pt)],
                        out_hbm.at[cid].at[pl.ds(sid * cap_pt, cap_pt)])

    return part_kernel


def _make_aggr_kernel(cap_chunks, nh, cl, d):
    rows_pt = nh // _NS
    cnt_pt = cl // _NS
    mesh = plsc.VectorSubcoreMesh(core_axis_name="c", subcore_axis_name="s")

    @functools.partial(
        pl.kernel,
        mesh=mesh,
        out_type=[
            jax.ShapeDtypeStruct((_NC, nh, d), jnp.float32),
            jax.ShapeDtypeStruct((_NC, cl), jnp.float32),
        ],
        scratch_types=[
            pltpu.VMEM((_IB, _K), jnp.int32),
            pltpu.VMEM((_IB, _K), jnp.int32),
            pltpu.VMEM((_IB, _K), jnp.int32),
            pltpu.VMEM((2, _K, d), jnp.float32),
            pltpu.VMEM((_K,), jnp.float32),
            pltpu.VMEM((16,), jnp.int32),
            pltpu.VMEM_SHARED((nh, d), jnp.float32),
            pltpu.VMEM_SHARED((nh, d), jnp.float32),
            pltpu.VMEM_SHARED((cl,), jnp.float32),
            pltpu.SemaphoreType.DMA,
            pltpu.SemaphoreType.DMA,
        ],
    )
    def aggr_kernel(hp_hbm, srcp_hbm, dstp_hbm, cntp_hbm, offc_hbm,
                    zrow_hbm, zcnt_hbm, aggr_out, cnt_out,
                    src_v, dst_v, cidx_v, rows_v, ones_v, off_v,
                    h_sh, aggr_sh, cnt_sh, sem0, sem1):
        sems = (sem0, sem1)
        cid = lax.axis_index("c")
        sid = lax.axis_index("s")
        pltpu.sync_copy(zrow_hbm.at[pl.ds(sid * rows_pt, rows_pt)],
                        aggr_sh.at[pl.ds(sid * rows_pt, rows_pt)])
        pltpu.sync_copy(zcnt_hbm.at[pl.ds(sid * cnt_pt, cnt_pt)],
                        cnt_sh.at[pl.ds(sid * cnt_pt, cnt_pt)])
        pltpu.sync_copy(offc_hbm, off_v)
        offv = off_v[...]
        offsc = [offv[k] for k in range(5)]
        for t in range(_K // _L):
            ones_v[pl.ds(t * _L, _L)] = jnp.ones((_L,), jnp.float32)

        for p in range(2):
            plsc.subcore_barrier()
            pltpu.sync_copy(hp_hbm.at[p].at[pl.ds(sid * rows_pt, rows_pt)],
                            h_sh.at[pl.ds(sid * rows_pt, rows_pt)])
            plsc.subcore_barrier()
            lo = jnp.where(cid == 0, offsc[p], offsc[2 + p])
            hi = jnp.where(cid == 0, offsc[p + 1], offsc[p + 3])
            per = (hi - lo) // 16
            my_lo = lo + sid * per

            def blk(b, carry):
                base = pl.multiple_of(my_lo + b * _IB, _IB)
                pltpu.sync_copy(srcp_hbm.at[pl.ds(base, _IB)], src_v)
                pltpu.sync_copy(dstp_hbm.at[pl.ds(base, _IB)], dst_v)
                pltpu.sync_copy(cntp_hbm.at[pl.ds(base, _IB)], cidx_v)
                pltpu.async_copy(h_sh.at[src_v.at[0]], rows_v.at[0], sems[0])
                for j in range(_IB):
                    if j + 1 < _IB:
                        pltpu.async_copy(h_sh.at[src_v.at[j + 1]],
                                         rows_v.at[(j + 1) % 2],
                                         sems[(j + 1) % 2])
                    pltpu.make_async_copy(h_sh.at[src_v.at[j]],
                                          rows_v.at[j % 2],
                                          sems[j % 2]).wait()
                    pltpu.sync_copy(rows_v.at[j % 2],
                                    aggr_sh.at[dst_v.at[j]], add=True)
                    pltpu.sync_copy(ones_v, cnt_sh.at[cidx_v.at[j]], add=True)
                return carry

            lax.fori_loop(0, per // _IB, blk, 0)

        plsc.subcore_barrier()
        pltpu.sync_copy(aggr_sh.at[pl.ds(sid * rows_pt, rows_pt)],
                        aggr_out.at[cid].at[pl.ds(sid * rows_pt, rows_pt)])
        pltpu.sync_copy(cnt_sh.at[pl.ds(sid * cnt_pt, cnt_pt)],
                        cnt_out.at[cid].at[pl.ds(sid * cnt_pt, cnt_pt)])

    return aggr_kernel


def kernel(x, edge_index, edge_attr, prelu_w, W_enc, emb1, emb2, W1, b1, W2, b2):
    n, d = x.shape
    e = edge_index.shape[1]
    f32, i32, u32 = jnp.float32, jnp.int32, jnp.uint32
    n2 = n // 2
    nh = -(-(n2 + 1) // (_NS * 8)) * (_NS * 8)
    cl = -(-((n2 + 1) * 18) // (_NS * 128)) * (_NS * 128)

    grid = n // _BN
    h = pl.pallas_call(
        _enc_body,
        grid=(grid,),
        in_specs=[
            pl.BlockSpec((1, 1), lambda i: (0, 0)),
            pl.BlockSpec((_BN, d), lambda i: (i, 0)),
            pl.BlockSpec((d, d), lambda i: (0, 0)),
        ],
        out_specs=pl.BlockSpec((_BN, d), lambda i: (i, 0)),
        out_shape=jax.ShapeDtypeStruct((n, d), f32),
    )(prelu_w.reshape(1, 1), x, W_enc.T)

    src = edge_index[0]
    dst = edge_index[1]
    code = edge_attr[:, 0] * 3 + edge_attr[:, 1]
    src_h = (src >= n2).astype(i32)
    dst_h = (dst >= n2).astype(i32)
    dst_l = dst - dst_h * n2
    rec = lax.bitcast_convert_type(
        (src.astype(u32) << 18) | (code.astype(u32) << 13)
        | dst_l.astype(u32), i32)
    grp = dst_h * 2 + src_h
    cum = [jnp.cumsum((grp == k).astype(i32)) for k in range(3)]
    rank3 = jnp.arange(e, dtype=i32) - cum[0] - cum[1] - cum[2]
    ranks = [cum[0] - 1, cum[1] - 1, cum[2] - 1, rank3]
    c012 = jnp.stack([cum[0][-1], cum[1][-1], cum[2][-1]])
    counts = jnp.concatenate([c012, (e - c012.sum())[None]])
    ends = jnp.cumsum(-(-counts // _ALIGN) * _ALIGN)
    offs = jnp.concatenate([jnp.zeros((1,), i32), ends.astype(i32)])
    pos = jnp.zeros((e,), i32)
    for k in range(4):
        pos = jnp.where(grp == k, ranks[k] + offs[k], pos)

    cap_total = e + 4 * _ALIGN
    cap_alloc = -(-(cap_total + _DUMP) // (_NS * 128)) * (_NS * 128)
    rec_trash = n2
    fill_cand = (offs[:4] + counts)[:, None] + jnp.arange(_ALIGN, dtype=i32)
    fill_valid = fill_cand < offs[1:][:, None]
    dump_ids = cap_total + (
        jnp.arange(4 * _ALIGN, dtype=i32) % _DUMP).reshape(4, _ALIGN)
    fill_pos = jnp.where(fill_valid, fill_cand, dump_ids).reshape(-1)

    n_entries = e + 4 * _ALIGN
    n_chunks_w = -(-n_entries // (_NW * _K * _IB)) * _IB
    e_pad_a = _NW * _K * n_chunks_w
    pad_a = e_pad_a - n_entries
    pos_all = jnp.concatenate(
        [pos, fill_pos,
         cap_total + jnp.arange(pad_a, dtype=i32) % _DUMP])
    rec_all = jnp.concatenate(
        [rec, jnp.full((4 * _ALIGN + pad_a,), rec_trash, i32)])
    posp = pos_all.reshape(_NW, n_chunks_w, _K)
    recp = rec_all.reshape(_NW, n_chunks_w, _K)

    ztrash = jnp.full((cap_alloc,), rec_trash, i32)
    recs2 = _make_part_kernel(n_chunks_w, cap_alloc)(posp, recp, ztrash)
    recs = jnp.where(recs2[0] != rec_trash, recs2[0], recs2[1])

    ur = lax.bitcast_convert_type(recs, u32)
    src_g = (ur >> 18).astype(i32)
    code_u = ((ur >> 13) & 31).astype(i32)
    dst_u = jnp.minimum((ur & 8191).astype(i32), n2)
    src_u = jnp.minimum(src_g - (src_g >= n2).astype(i32) * n2, nh - 1)
    cidx_u = dst_u * 18 + code_u
    cap_chunks = cap_alloc // _K
    srcp2 = src_u.reshape(cap_chunks, _K)
    dstp2 = dst_u.reshape(cap_chunks, _K)
    cntp2 = cidx_u.reshape(cap_chunks, _K)
    offc = jnp.concatenate([offs // _K, jnp.zeros((11,), i32)])

    hp = jnp.zeros((2, nh, d), f32)
    hp = hp.at[0, :n2].set(lax.slice(h, (0, 0), (n2, d)))
    hp = hp.at[1, :n2].set(lax.slice(h, (n2, 0), (n, d)))
    zrow = jnp.zeros((nh, d), f32)
    zcnt = jnp.zeros((cl,), f32)

    aggr2, cnt2 = _make_aggr_kernel(cap_chunks, nh, cl, d)(
        hp, srcp2, dstp2, cntp2, offc, zrow, zcnt)

    c0 = lax.slice(cnt2[0], (0,), (n2 * 18,)).reshape(n2, 18)
    c1 = lax.slice(cnt2[1], (0,), (n2 * 18,)).reshape(n2, 18)
    call = jnp.concatenate([c0, c1])

    emb18 = (emb1[:, None, :] + emb2[None, :, :]).reshape(18, d)
    nb_half = n2 // _BN

    out = pl.pallas_call(
        _mlp_body,
        grid=(grid,),
        in_specs=[
            pl.BlockSpec((1, _BN, d),
                         lambda i: (i // nb_half, i % nb_half, 0)),
            pl.BlockSpec((_BN, d), lambda i: (i, 0)),
            pl.BlockSpec((_BN, 18), lambda i: (i, 0)),
            pl.BlockSpec((18, d), lambda i: (0, 0)),
            pl.BlockSpec((d, 2 * d), lambda i: (0, 0)),
            pl.BlockSpec((1, 2 * d), lambda i: (0, 0)),
            pl.BlockSpec((2 * d, d), lambda i: (0, 0)),
            pl.BlockSpec((1, d), lambda i: (0, 0)),
        ],
        out_specs=pl.BlockSpec((_BN, d), lambda i: (i, 0)),
        out_shape=jax.ShapeDtypeStruct((n, d), f32),
    )(aggr2, h, call, emb18, W1.T, b1.reshape(1, -1), W2.T,
      b2.reshape(1, -1))
    return out
